# baseline (speedup 1.0000x reference)
import functools

import jax
import jax.numpy as jnp
from jax.experimental import pallas as pl


def _pos_body(col_ref, row_ref, out_ref):
    col = col_ref[...]
    row = row_ref[...]
    col_t = col.T
    row_t = row.T
    x_flat = jnp.reshape(
        jnp.broadcast_to(col_t[:, None, :], (128, 32, 32)), (128, 1024)
    )
    y_flat = jnp.reshape(
        jnp.broadcast_to(row_t[:, :, None], (128, 32, 32)), (128, 1024)
    )
    pos = jnp.concatenate([x_flat, y_flat], axis=0)
    out_ref[...] = jnp.broadcast_to(pos[None], (4, 256, 1024))


@functools.partial(jax.jit, static_argnames=("interpret",))
def _pos_embed(row_embed, col_embed, interpret=False):
    out = pl.pallas_call(
        _pos_body,
        out_shape=jax.ShapeDtypeStruct((4, 256, 1024), jnp.float32),
        interpret=interpret,
    )(col_embed[:32], row_embed[:32])
    return out.reshape(4, 256, 32, 32)


def kernel(x, row_embed, col_embed):
    del x
    return _pos_embed(row_embed, col_embed)


# speedup vs baseline: 1.0423x; 1.0423x over previous
import functools

import jax
import jax.numpy as jnp
from jax.experimental import pallas as pl
from jax.experimental.pallas import tpu as pltpu


def _pos_body(col_ref, row_ref, out_ref, acc_ref):
    @pl.when(pl.program_id(0) == 0)
    def _():
        sub = jax.lax.broadcasted_iota(jnp.int32, (32, 1024), 0)
        lane = jax.lax.broadcasted_iota(jnp.int32, (32, 1024), 1)
        s_col = (lane % 32 == sub).astype(jnp.float32)
        s_row = (lane // 32 == sub).astype(jnp.float32)
        x_flat = jax.lax.dot_general(
            col_ref[...], s_col, (((0,), (0,)), ((), ())),
            preferred_element_type=jnp.float32)
        y_flat = jax.lax.dot_general(
            row_ref[...], s_row, (((0,), (0,)), ((), ())),
            preferred_element_type=jnp.float32)
        acc_ref[:128] = x_flat
        acc_ref[128:] = y_flat

    out_ref[0] = acc_ref[...]


@functools.partial(jax.jit, static_argnames=("interpret",))
def _pos_embed(row_embed, col_embed, interpret=False):
    out = pl.pallas_call(
        _pos_body,
        grid=(4,),
        in_specs=[
            pl.BlockSpec((32, 128), lambda i: (0, 0)),
            pl.BlockSpec((32, 128), lambda i: (0, 0)),
        ],
        out_specs=pl.BlockSpec((1, 256, 1024), lambda i: (i, 0, 0)),
        out_shape=jax.ShapeDtypeStruct((4, 256, 1024), jnp.float32),
        scratch_shapes=[pltpu.VMEM((256, 1024), jnp.float32)],
        interpret=interpret,
    )(col_embed[:32], row_embed[:32])
    return out.reshape(4, 256, 32, 32)


def kernel(x, row_embed, col_embed):
    del x
    return _pos_embed(row_embed, col_embed)
